# baseline (speedup 1.0000x reference)
import functools

import jax
import jax.numpy as jnp
from jax import lax
from jax.experimental import pallas as pl
from jax.experimental.pallas import tpu as pltpu
from jax.experimental.pallas import tpu_sc as plsc

VOCAB = 1000000
EMBED = 64
MAXLEN = 512
BATCH = 4096
SEQ = 200

NC = 2
NS = 16
NW = NC * NS

BB = BATCH // 128
SB = SEQ // 8
LANES = 16
VPR = EMBED // LANES


def _body(x4_hbm, tok_hbm, pos_hbm, out_hbm, idx_v, g0, g1, t0, t1, pos_v,
          sg0, sg1, sw0, sw1):
    g = (g0, g1)
    t = (t0, t1)
    sg = (sg0, sg1)
    sw = (sw0, sw1)
    w = lax.axis_index("s") * NC + lax.axis_index("c")

    pltpu.sync_copy(x4_hbm.at[:, pl.ds(w, 1)], idx_v)
    pltpu.sync_copy(pos_hbm.at[pl.ds(0, SEQ)], pos_v)

    iota = lax.iota(jnp.int32, LANES)
    idx_j = [iota + (k * LANES) for k in range(VPR)]

    def fire_gather(s, b):
        pltpu.async_copy(
            tok_hbm.at[idx_v.at[s // 8, 0, lax.rem(s, 8)]], g[b], sg[b]
        )

    def drain_gather(s, b):
        pltpu.make_async_copy(
            tok_hbm.at[idx_v.at[s // 8, 0, lax.rem(s, 8)]], g[b], sg[b]
        ).wait()

    def transpose_add(s, b):
        pos_k = [pos_v[s, pl.ds(k * LANES, LANES)] for k in range(VPR)]

        def row(r, carry):
            rb = jnp.broadcast_to(r, (LANES,)).astype(jnp.int32)
            for k in range(VPR):
                v = g[b][r, pl.ds(k * LANES, LANES)] + pos_k[k]
                plsc.store_scatter(t[b], [idx_j[k], rb], v)
            return carry

        lax.fori_loop(0, 128, row, 0)

    def fire_write(s, b):
        for jb in range(8):
            pltpu.async_copy(
                t[b].at[pl.ds(jb * 8, 8)], out_hbm.at[s, jb, w], sw[b]
            )

    def drain_write(s, b):
        for jb in range(8):
            pltpu.make_async_copy(
                t[b].at[pl.ds(jb * 8, 8)], out_hbm.at[s, jb, w], sw[b]
            ).wait()

    fire_gather(0, 0)

    def pair(i, carry):
        s0 = 2 * i
        for b in range(2):
            s = s0 + b
            nb = 1 - b

            @pl.when(s + 1 < SEQ)
            def _():
                fire_gather(s + 1, nb)

            drain_gather(s, b)

            @pl.when(s >= 2)
            def _():
                drain_write(s - 2, b)

            transpose_add(s, b)
            fire_write(s, b)
        return carry

    lax.fori_loop(0, SEQ // 2, pair, 0)
    drain_write(SEQ - 2, 0)
    drain_write(SEQ - 1, 1)


def kernel(x, token_table, pos_table):
    x4 = x.T.reshape(SB, 8, BB, 128).transpose(0, 2, 1, 3)
    tok128 = jnp.pad(token_table, ((0, 0), (0, 128 - EMBED)))

    mesh = plsc.VectorSubcoreMesh(core_axis_name="c", subcore_axis_name="s")
    run = functools.partial(
        pl.kernel,
        mesh=mesh,
        out_type=jax.ShapeDtypeStruct((SEQ, 8, BB, 8, 128), jnp.float32),
        scratch_types=[
            pltpu.VMEM((SB, 1, 8, 128), jnp.int32),
            pltpu.VMEM((128, 128), jnp.float32),
            pltpu.VMEM((128, 128), jnp.float32),
            pltpu.VMEM((64, 128), jnp.float32),
            pltpu.VMEM((64, 128), jnp.float32),
            pltpu.VMEM((SEQ, EMBED), jnp.float32),
            pltpu.SemaphoreType.DMA,
            pltpu.SemaphoreType.DMA,
            pltpu.SemaphoreType.DMA,
            pltpu.SemaphoreType.DMA,
        ],
        compiler_params=pltpu.CompilerParams(
            use_tc_tiling_on_sc=False, needs_layout_passes=False
        ),
    )(_body)
    out5 = run(x4, tok128, pos_table)
    return out5.transpose(2, 4, 0, 1, 3).reshape(BATCH, SEQ, EMBED)


# speedup vs baseline: 1.5948x; 1.5948x over previous
import functools

import jax
import jax.numpy as jnp
from jax import lax
from jax.experimental import pallas as pl
from jax.experimental.pallas import tpu as pltpu
from jax.experimental.pallas import tpu_sc as plsc

VOCAB = 1000000
EMBED = 64
MAXLEN = 512
BATCH = 4096
SEQ = 200

NC = 2
NS = 16
NW = NC * NS

BB = BATCH // 128
SB = SEQ // 8
LANES = 16
VPR = EMBED // LANES


def _body(x4_hbm, tok_hbm, pos_hbm, out_hbm, idx_v, g0, g1, t0, t1, pos_v,
          sg0, sg1, sw0, sw1):
    g = (g0, g1)
    t = (t0, t1)
    sg = (sg0, sg1)
    sw = (sw0, sw1)
    w = lax.axis_index("s") * NC + lax.axis_index("c")

    pltpu.sync_copy(x4_hbm.at[:, pl.ds(w, 1)], idx_v)
    pltpu.sync_copy(pos_hbm.at[pl.ds(0, SEQ)], pos_v)

    iota = lax.iota(jnp.int32, LANES)
    idx_j = [iota + (k * LANES) for k in range(VPR)]

    def fire_gather(s, b):
        pltpu.async_copy(
            tok_hbm.at[idx_v.at[s // 8, 0, lax.rem(s, 8)]], g[b], sg[b]
        )

    def drain_gather(s, b):
        pltpu.make_async_copy(
            tok_hbm.at[idx_v.at[s // 8, 0, lax.rem(s, 8)]], g[b], sg[b]
        ).wait()

    def transpose_add(s, b):
        pos_k = [pos_v[s, pl.ds(k * LANES, LANES)] for k in range(VPR)]

        def row2(i, carry):
            r0 = 2 * i
            for u in range(2):
                r = r0 + u
                rb = jnp.broadcast_to(r, (LANES,)).astype(jnp.int32)
                for k in range(VPR):
                    v = g[b][r, pl.ds(k * LANES, LANES)] + pos_k[k]
                    plsc.store_scatter(t[b], [idx_j[k], rb], v)
            return carry

        lax.fori_loop(0, 64, row2, 0)

    def fire_write(s, b):
        for jb in range(8):
            pltpu.async_copy(
                t[b].at[pl.ds(jb * 8, 8), pl.ds(0, 128)],
                out_hbm.at[s, jb, w],
                sw[b],
            )

    def drain_write(s, b):
        for jb in range(8):
            pltpu.make_async_copy(
                t[b].at[pl.ds(jb * 8, 8), pl.ds(0, 128)],
                out_hbm.at[s, jb, w],
                sw[b],
            ).wait()

    fire_gather(0, 0)

    def pair(i, carry):
        s0 = 2 * i
        for b in range(2):
            s = s0 + b
            nb = 1 - b

            @pl.when(s + 1 < SEQ)
            def _():
                fire_gather(s + 1, nb)

            drain_gather(s, b)

            @pl.when(s >= 2)
            def _():
                drain_write(s - 2, b)

            transpose_add(s, b)
            fire_write(s, b)
        return carry

    lax.fori_loop(0, SEQ // 2, pair, 0)
    drain_write(SEQ - 2, 0)
    drain_write(SEQ - 1, 1)


def kernel(x, token_table, pos_table):
    x4 = x.T.reshape(SB, 8, BB, 128).transpose(0, 2, 1, 3)
    tok128 = jnp.pad(token_table, ((0, 0), (0, 128 - EMBED)))

    mesh = plsc.VectorSubcoreMesh(core_axis_name="c", subcore_axis_name="s")
    run = functools.partial(
        pl.kernel,
        mesh=mesh,
        out_type=jax.ShapeDtypeStruct((SEQ, 8, BB, 8, 128), jnp.float32),
        scratch_types=[
            pltpu.VMEM((SB, 1, 8, 128), jnp.int32),
            pltpu.VMEM((128, 128), jnp.float32),
            pltpu.VMEM((128, 128), jnp.float32),
            pltpu.VMEM((64, 133), jnp.float32),
            pltpu.VMEM((64, 133), jnp.float32),
            pltpu.VMEM((SEQ, EMBED), jnp.float32),
            pltpu.SemaphoreType.DMA,
            pltpu.SemaphoreType.DMA,
            pltpu.SemaphoreType.DMA,
            pltpu.SemaphoreType.DMA,
        ],
        compiler_params=pltpu.CompilerParams(
            use_tc_tiling_on_sc=False, needs_layout_passes=False
        ),
    )(_body)
    out5 = run(x4, tok128, pos_table)
    return out5.transpose(2, 4, 0, 1, 3).reshape(BATCH, SEQ, EMBED)


# speedup vs baseline: 1.5953x; 1.0003x over previous
import functools

import jax
import jax.numpy as jnp
from jax import lax
from jax.experimental import pallas as pl
from jax.experimental.pallas import tpu as pltpu
from jax.experimental.pallas import tpu_sc as plsc

VOCAB = 1000000
EMBED = 64
MAXLEN = 512
BATCH = 4096
SEQ = 200

NC = 2
NS = 16
NW = NC * NS

BB = BATCH // 128
SB = SEQ // 8
LANES = 16
VPR = EMBED // LANES


def _body(x4_hbm, tok_hbm, pos_hbm, out_hbm, idx_v, g0, g1, t0, t1, pos_v,
          sg0, sg1, sw0, sw1):
    g = (g0, g1)
    t = (t0, t1)
    sg = (sg0, sg1)
    sw = (sw0, sw1)
    w = lax.axis_index("s") * NC + lax.axis_index("c")

    pltpu.sync_copy(x4_hbm.at[:, pl.ds(w, 1)], idx_v)
    pltpu.sync_copy(pos_hbm.at[pl.ds(0, SEQ)], pos_v)

    def dbl(sb, carry):
        for sl in range(8):
            for k in range(8):
                cs = pl.ds(k * LANES, LANES)
                idx_v[sb, 0, sl, cs] = idx_v[sb, 0, sl, cs] * 2
        return carry

    lax.fori_loop(0, SB, dbl, 0)

    iota = lax.iota(jnp.int32, LANES)
    idx_j = [iota + (k * LANES) for k in range(VPR)]

    def fire_gather(s, b):
        pltpu.async_copy(
            tok_hbm.at[idx_v.at[s // 8, 0, lax.rem(s, 8)]], g[b], sg[b]
        )

    def drain_gather(s, b):
        pltpu.make_async_copy(
            tok_hbm.at[idx_v.at[s // 8, 0, lax.rem(s, 8)]], g[b], sg[b]
        ).wait()

    def transpose_add(s, b):
        pos_k = [pos_v[s, pl.ds(k * LANES, LANES)] for k in range(VPR)]

        def row2(i, carry):
            r0 = 2 * i
            for u in range(2):
                r = r0 + u
                rb = jnp.broadcast_to(r, (LANES,)).astype(jnp.int32)
                for k in range(VPR):
                    v = g[b][r, pl.ds(k * LANES, LANES)] + pos_k[k]
                    plsc.store_scatter(t[b], [idx_j[k], rb], v)
            return carry

        lax.fori_loop(0, 64, row2, 0)

    def fire_write(s, b):
        for jb in range(8):
            pltpu.async_copy(
                t[b].at[pl.ds(jb * 8, 8), pl.ds(0, 128)],
                out_hbm.at[s, jb, w],
                sw[b],
            )

    def drain_write(s, b):
        for jb in range(8):
            pltpu.make_async_copy(
                t[b].at[pl.ds(jb * 8, 8), pl.ds(0, 128)],
                out_hbm.at[s, jb, w],
                sw[b],
            ).wait()

    fire_gather(0, 0)

    def pair(i, carry):
        s0 = 2 * i
        for b in range(2):
            s = s0 + b
            nb = 1 - b

            @pl.when(s + 1 < SEQ)
            def _():
                fire_gather(s + 1, nb)

            drain_gather(s, b)

            @pl.when(s >= 2)
            def _():
                drain_write(s - 2, b)

            transpose_add(s, b)
            fire_write(s, b)
        return carry

    lax.fori_loop(0, SEQ // 2, pair, 0)
    drain_write(SEQ - 2, 0)
    drain_write(SEQ - 1, 1)


def kernel(x, token_table, pos_table):
    x4 = x.T.reshape(SB, 8, BB, 128).transpose(0, 2, 1, 3)
    tokv = jnp.pad(token_table, ((0, 0), (0, 128 - EMBED))).reshape(
        2 * VOCAB, EMBED
    )

    mesh = plsc.VectorSubcoreMesh(core_axis_name="c", subcore_axis_name="s")
    run = functools.partial(
        pl.kernel,
        mesh=mesh,
        out_type=jax.ShapeDtypeStruct((SEQ, 8, BB, 8, 128), jnp.float32),
        scratch_types=[
            pltpu.VMEM((SB, 1, 8, 128), jnp.int32),
            pltpu.VMEM((128, EMBED), jnp.float32),
            pltpu.VMEM((128, EMBED), jnp.float32),
            pltpu.VMEM((64, 133), jnp.float32),
            pltpu.VMEM((64, 133), jnp.float32),
            pltpu.VMEM((SEQ, EMBED), jnp.float32),
            pltpu.SemaphoreType.DMA,
            pltpu.SemaphoreType.DMA,
            pltpu.SemaphoreType.DMA,
            pltpu.SemaphoreType.DMA,
        ],
        compiler_params=pltpu.CompilerParams(
            use_tc_tiling_on_sc=False, needs_layout_passes=False
        ),
    )(_body)
    out5 = run(x4, tokv, pos_table)
    return out5.transpose(2, 4, 0, 1, 3).reshape(BATCH, SEQ, EMBED)


# speedup vs baseline: 1.6293x; 1.0213x over previous
import functools

import jax
import jax.numpy as jnp
from jax import lax
from jax.experimental import pallas as pl
from jax.experimental.pallas import tpu as pltpu
from jax.experimental.pallas import tpu_sc as plsc

VOCAB = 1000000
EMBED = 64
MAXLEN = 512
BATCH = 4096
SEQ = 200

NC = 2
NS = 16
NW = NC * NS

BB = BATCH // 128
SB = SEQ // 8
LANES = 16
VPR = EMBED // LANES


def _body(x4_hbm, tok_hbm, pos_hbm, out_hbm, idx_v, g0, g1, t0, t1, pos_v,
          sg0, sg1, sw0, sw1):
    g = (g0, g1)
    t = (t0, t1)
    sg = (sg0, sg1)
    sw = (sw0, sw1)
    w = lax.axis_index("s") * NC + lax.axis_index("c")

    pltpu.sync_copy(x4_hbm.at[:, pl.ds(w, 1)], idx_v)
    pltpu.sync_copy(pos_hbm.at[pl.ds(0, SEQ)], pos_v)

    def dbl(sb, carry):
        for sl in range(8):
            for k in range(8):
                cs = pl.ds(k * LANES, LANES)
                idx_v[sb, 0, sl, cs] = idx_v[sb, 0, sl, cs] * 2
        return carry

    lax.fori_loop(0, SB, dbl, 0)

    iota = lax.iota(jnp.int32, LANES)
    idx_hi = [(iota + (k * LANES)) >> 3 for k in range(VPR)]
    idx_lo = [(iota + (k * LANES)) & 7 for k in range(VPR)]
    zero_v = jnp.zeros((LANES,), jnp.int32)

    def fire_gather(s, b):
        pltpu.async_copy(
            tok_hbm.at[idx_v.at[s // 8, 0, lax.rem(s, 8)]], g[b], sg[b]
        )

    def drain_gather(s, b):
        pltpu.make_async_copy(
            tok_hbm.at[idx_v.at[s // 8, 0, lax.rem(s, 8)]], g[b], sg[b]
        ).wait()

    def transpose_add(s, b):
        pos_k = [pos_v[s, pl.ds(k * LANES, LANES)] for k in range(VPR)]

        def row4(i, carry):
            r0 = 4 * i
            for u in range(4):
                r = r0 + u
                rb = jnp.broadcast_to(r, (LANES,)).astype(jnp.int32)
                for k in range(VPR):
                    v = g[b][r, pl.ds(k * LANES, LANES)] + pos_k[k]
                    plsc.store_scatter(
                        t[b], [idx_hi[k], zero_v, idx_lo[k], rb], v
                    )
            return carry

        lax.fori_loop(0, 32, row4, 0)

    def fire_write(s, b):
        pltpu.async_copy(
            t[b].at[:, :, :, pl.ds(0, 128)],
            out_hbm.at[s, :, pl.ds(w, 1)],
            sw[b],
        )

    def drain_write(s, b):
        pltpu.make_async_copy(
            t[b].at[:, :, :, pl.ds(0, 128)],
            out_hbm.at[s, :, pl.ds(w, 1)],
            sw[b],
        ).wait()

    fire_gather(0, 0)

    def pair(i, carry):
        s0 = 2 * i
        for b in range(2):
            s = s0 + b
            nb = 1 - b

            @pl.when(s + 1 < SEQ)
            def _():
                fire_gather(s + 1, nb)

            drain_gather(s, b)

            @pl.when(s >= 2)
            def _():
                drain_write(s - 2, b)

            transpose_add(s, b)
            fire_write(s, b)
        return carry

    lax.fori_loop(0, SEQ // 2, pair, 0)
    drain_write(SEQ - 2, 0)
    drain_write(SEQ - 1, 1)


def kernel(x, token_table, pos_table):
    x4 = x.T.reshape(SB, 8, BB, 128).transpose(0, 2, 1, 3)
    tokv = jnp.pad(token_table, ((0, 0), (0, 128 - EMBED))).reshape(
        2 * VOCAB, EMBED
    )

    mesh = plsc.VectorSubcoreMesh(core_axis_name="c", subcore_axis_name="s")
    run = functools.partial(
        pl.kernel,
        mesh=mesh,
        out_type=jax.ShapeDtypeStruct((SEQ, 8, BB, 8, 128), jnp.float32),
        scratch_types=[
            pltpu.VMEM((SB, 1, 8, 128), jnp.int32),
            pltpu.VMEM((128, EMBED), jnp.float32),
            pltpu.VMEM((128, EMBED), jnp.float32),
            pltpu.VMEM((8, 1, 8, 133), jnp.float32),
            pltpu.VMEM((8, 1, 8, 133), jnp.float32),
            pltpu.VMEM((SEQ, EMBED), jnp.float32),
            pltpu.SemaphoreType.DMA,
            pltpu.SemaphoreType.DMA,
            pltpu.SemaphoreType.DMA,
            pltpu.SemaphoreType.DMA,
        ],
        compiler_params=pltpu.CompilerParams(
            use_tc_tiling_on_sc=False, needs_layout_passes=False
        ),
    )(_body)
    out5 = run(x4, tokv, pos_table)
    return out5.transpose(2, 4, 0, 1, 3).reshape(BATCH, SEQ, EMBED)
